# baseline (speedup 1.0000x reference)
import functools

import jax
import jax.numpy as jnp
from jax import lax
from jax.experimental import pallas as pl
from jax.experimental.pallas import tpu as pltpu
from jax.experimental.pallas import tpu_sc as plsc

N_ROWS = 16384
N_COLS = 16384
NNZ = 2684354
D = 64

NC = 2
NS = 16
NW = NC * NS
K = 128
CHUNKS_PER_TILE = 656
M = NW * CHUNKS_PER_TILE
NNZ_PAD = M * K

_mesh = plsc.VectorSubcoreMesh(core_axis_name="c", subcore_axis_name="s")


@functools.partial(
    pl.kernel,
    out_type=jax.ShapeDtypeStruct((NW, N_ROWS // NS, D), jnp.float32),
    mesh=_mesh,
    compiler_params=pltpu.CompilerParams(
        needs_layout_passes=False, use_tc_tiling_on_sc=False),
    scratch_types=[
        pltpu.VMEM((1, K), jnp.int32),
        pltpu.VMEM((1, K), jnp.int32),
        pltpu.VMEM((K + 16,), jnp.float32),
        pltpu.VMEM((K, D), jnp.float32),
        pltpu.VMEM_SHARED((N_ROWS, D), jnp.float32),
        pltpu.SemaphoreType.DMA,
    ],
)
def _sc_spmm(table, gidx, sidx, vals, zeros, out,
             gidx_v, sidx_v, vals_v, rows_v, acc, sem):
    cid = lax.axis_index("c")
    sid = lax.axis_index("s")
    wid = cid * NS + sid
    rpt = N_ROWS // NS

    pltpu.sync_copy(zeros.at[pl.ds(sid * rpt, rpt)],
                    acc.at[pl.ds(sid * rpt, rpt)])
    plsc.subcore_barrier()

    def body(i, carry):
        c = wid * CHUNKS_PER_TILE + i
        pltpu.sync_copy(gidx.at[pl.ds(c, 1)], gidx_v)
        pltpu.sync_copy(sidx.at[pl.ds(c, 1)], sidx_v)
        pltpu.sync_copy(vals.at[pl.ds(c * K, K)], vals_v.at[pl.ds(16, K)])
        pltpu.async_copy(table.at[gidx_v.at[0]], rows_v, sem).wait()
        for r in range(K):
            b = plsc.load_gather(vals_v, [jnp.full((16,), 16 + r, jnp.int32)])
            for j in range(D // 16):
                sl = pl.ds(j * 16, 16)
                rows_v[r, sl] = rows_v[r, sl] * b
        pltpu.sync_copy(rows_v, acc.at[sidx_v.at[0]], add=True)
        return carry

    lax.fori_loop(0, CHUNKS_PER_TILE, body, 0)
    plsc.subcore_barrier()
    pltpu.sync_copy(acc.at[pl.ds(sid * rpt, rpt)], out.at[wid])


_BLK = 1024


def _div_body(sino_ref, pp_ref, out_ref):
    out_ref[...] = sino_ref[...] / (pp_ref[0] + pp_ref[1] + 1e-8)


def _temp_proj(sinogram, pp):
    return pl.pallas_call(
        _div_body,
        grid=(N_ROWS // _BLK,),
        in_specs=[
            pl.BlockSpec((_BLK, D), lambda i: (i, 0)),
            pl.BlockSpec((2, _BLK, D), lambda i: (0, i, 0)),
        ],
        out_specs=pl.BlockSpec((_BLK, D), lambda i: (i, 0)),
        out_shape=jax.ShapeDtypeStruct((N_ROWS, D), jnp.float32),
    )(sinogram, pp)


def _final_body(img_ref, eff_ref, pp_ref, out_ref):
    out_ref[...] = img_ref[...] / eff_ref[...] * (pp_ref[0] + pp_ref[1])


def _final(image, eff, pp):
    return pl.pallas_call(
        _final_body,
        grid=(N_COLS // _BLK,),
        in_specs=[
            pl.BlockSpec((_BLK, D), lambda i: (i, 0)),
            pl.BlockSpec((_BLK, D), lambda i: (i, 0)),
            pl.BlockSpec((2, _BLK, D), lambda i: (0, i, 0)),
        ],
        out_specs=pl.BlockSpec((_BLK, D), lambda i: (i, 0)),
        out_shape=jax.ShapeDtypeStruct((N_COLS, D), jnp.float32),
    )(image, eff, pp)


def kernel(image, efficiency_map, sinogram, matrix_vals, matrix_rows, matrix_cols):
    pad = NNZ_PAD - NNZ
    cols2 = jnp.concatenate([matrix_cols, jnp.zeros((pad,), jnp.int32)]).reshape(M, K)
    rows2 = jnp.concatenate([matrix_rows, jnp.zeros((pad,), jnp.int32)]).reshape(M, K)
    vals2 = jnp.concatenate([matrix_vals, jnp.zeros((pad,), jnp.float32)])
    zeros = jnp.zeros((N_ROWS, D), jnp.float32)

    pp = _sc_spmm(image, cols2, rows2, vals2, zeros).reshape(NC, N_ROWS, D)
    temp = _temp_proj(sinogram, pp)
    bp = _sc_spmm(temp, rows2, cols2, vals2, zeros).reshape(NC, N_COLS, D)
    return _final(image, efficiency_map, bp)


# speedup vs baseline: 1.2051x; 1.2051x over previous
import functools

import jax
import jax.numpy as jnp
from jax import lax
from jax.experimental import pallas as pl
from jax.experimental.pallas import tpu as pltpu
from jax.experimental.pallas import tpu_sc as plsc

N_ROWS = 16384
N_COLS = 16384
NNZ = 2684354
D = 64

NC = 2
NS = 16
NW = NC * NS
K = 128
CHUNKS_PER_TILE = 656
M = NW * CHUNKS_PER_TILE
NNZ_PAD = M * K

_mesh = plsc.VectorSubcoreMesh(core_axis_name="c", subcore_axis_name="s")


@functools.partial(
    pl.kernel,
    out_type=jax.ShapeDtypeStruct((NW, N_ROWS // NS, D), jnp.float32),
    mesh=_mesh,
    compiler_params=pltpu.CompilerParams(
        needs_layout_passes=False, use_tc_tiling_on_sc=False),
    scratch_types=[
        pltpu.VMEM((3, K), jnp.int32),
        pltpu.VMEM((3, K), jnp.int32),
        pltpu.VMEM((K, D), jnp.float32),
        pltpu.VMEM((K, D), jnp.float32),
        pltpu.VMEM_SHARED((N_ROWS, D), jnp.float32),
        pltpu.SemaphoreType.DMA,
        pltpu.SemaphoreType.DMA,
        pltpu.SemaphoreType.DMA,
        pltpu.SemaphoreType.DMA,
    ],
)
def _sc_spmm(table, pk, zeros, out,
             pk0, pk1, rows0, rows1, acc, sg0, sg1, ss0, ss1):
    cid = lax.axis_index("c")
    sid = lax.axis_index("s")
    wid = cid * NS + sid
    rpt = N_ROWS // NS
    base = wid * CHUNKS_PER_TILE

    pltpu.sync_copy(zeros.at[pl.ds(sid * rpt, rpt)],
                    acc.at[pl.ds(sid * rpt, rpt)])
    plsc.subcore_barrier()

    pks = (pk0, pk1)
    rowss = (rows0, rows1)
    sgs = (sg0, sg1)
    sss = (ss0, ss1)

    def scale(pkb, rowsb):
        for r in range(K):
            bi = plsc.load_gather(pkb, [jnp.full((16,), 2, jnp.int32),
                                        jnp.full((16,), r, jnp.int32)])
            bv = plsc.bitcast(bi, jnp.float32)
            for j in range(D // 16):
                sl = pl.ds(j * 16, 16)
                rowsb[r, sl] = rowsb[r, sl] * bv

    pltpu.sync_copy(pk.at[base], pk0)
    pltpu.async_copy(table.at[pk0.at[0]], rows0, sg0)

    def body(jj, carry):
        for b in range(2):
            t = jj * 2 + b
            o = 1 - b
            pltpu.make_async_copy(zeros.at[pl.ds(0, K)], rowss[b], sgs[b]).wait()

            @pl.when(t > 0)
            def _():
                pltpu.make_async_copy(zeros.at[pl.ds(0, K)], rowss[o], sss[o]).wait()

            @pl.when(t + 1 < CHUNKS_PER_TILE)
            def _():
                pltpu.sync_copy(pk.at[base + t + 1], pks[o])
                pltpu.async_copy(table.at[pks[o].at[0]], rowss[o], sgs[o])

            scale(pks[b], rowss[b])
            pltpu.async_copy(rowss[b], acc.at[pks[b].at[1]], sss[b], add=True)
        return carry

    lax.fori_loop(0, CHUNKS_PER_TILE // 2, body, 0)
    pltpu.make_async_copy(zeros.at[pl.ds(0, K)], rows1, ss1).wait()
    plsc.subcore_barrier()
    pltpu.sync_copy(acc.at[pl.ds(sid * rpt, rpt)], out.at[wid])


_BLK = 1024


def _div_body(sino_ref, pp_ref, out_ref):
    out_ref[...] = sino_ref[...] / (pp_ref[0] + pp_ref[1] + 1e-8)


def _temp_proj(sinogram, pp):
    return pl.pallas_call(
        _div_body,
        grid=(N_ROWS // _BLK,),
        in_specs=[
            pl.BlockSpec((_BLK, D), lambda i: (i, 0)),
            pl.BlockSpec((2, _BLK, D), lambda i: (0, i, 0)),
        ],
        out_specs=pl.BlockSpec((_BLK, D), lambda i: (i, 0)),
        out_shape=jax.ShapeDtypeStruct((N_ROWS, D), jnp.float32),
    )(sinogram, pp)


def _final_body(img_ref, eff_ref, pp_ref, out_ref):
    out_ref[...] = img_ref[...] / eff_ref[...] * (pp_ref[0] + pp_ref[1])


def _final(image, eff, pp):
    return pl.pallas_call(
        _final_body,
        grid=(N_COLS // _BLK,),
        in_specs=[
            pl.BlockSpec((_BLK, D), lambda i: (i, 0)),
            pl.BlockSpec((_BLK, D), lambda i: (i, 0)),
            pl.BlockSpec((2, _BLK, D), lambda i: (0, i, 0)),
        ],
        out_specs=pl.BlockSpec((_BLK, D), lambda i: (i, 0)),
        out_shape=jax.ShapeDtypeStruct((N_COLS, D), jnp.float32),
    )(image, eff, pp)


def kernel(image, efficiency_map, sinogram, matrix_vals, matrix_rows, matrix_cols):
    pad = NNZ_PAD - NNZ
    cols2 = jnp.concatenate([matrix_cols, jnp.zeros((pad,), jnp.int32)]).reshape(M, K)
    rows2 = jnp.concatenate([matrix_rows, jnp.zeros((pad,), jnp.int32)]).reshape(M, K)
    valsb = jax.lax.bitcast_convert_type(
        jnp.concatenate([matrix_vals, jnp.zeros((pad,), jnp.float32)]),
        jnp.int32).reshape(M, K)
    pk_f = jnp.stack([cols2, rows2, valsb], axis=1)
    pk_b = jnp.stack([rows2, cols2, valsb], axis=1)
    zeros = jnp.zeros((N_ROWS, D), jnp.float32)

    pp = _sc_spmm(image, pk_f, zeros).reshape(NC, N_ROWS, D)
    temp = _temp_proj(sinogram, pp)
    bp = _sc_spmm(temp, pk_b, zeros).reshape(NC, N_COLS, D)
    return _final(image, efficiency_map, bp)


# speedup vs baseline: 2.2101x; 1.8339x over previous
import functools

import jax
import jax.numpy as jnp
from jax import lax
from jax.experimental import pallas as pl
from jax.experimental.pallas import tpu as pltpu
from jax.experimental.pallas import tpu_sc as plsc

N_ROWS = 16384
N_COLS = 16384
NNZ = 2684354
D = 64

NC = 2
NS = 16
NW = NC * NS
K = 128
CHUNKS_PER_TILE = 656
M = NW * CHUNKS_PER_TILE
NNZ_PAD = M * K

_mesh = plsc.VectorSubcoreMesh(core_axis_name="c", subcore_axis_name="s")


@functools.partial(
    pl.kernel,
    out_type=jax.ShapeDtypeStruct((NW, N_ROWS // NS, D), jnp.float32),
    mesh=_mesh,
    compiler_params=pltpu.CompilerParams(
        needs_layout_passes=False, use_tc_tiling_on_sc=False),
    scratch_types=[
        pltpu.VMEM((3, K), jnp.int32),
        pltpu.VMEM((3, K), jnp.int32),
        pltpu.VMEM((K, D), jnp.float32),
        pltpu.VMEM((K, D), jnp.float32),
        pltpu.VMEM_SHARED((N_ROWS, D), jnp.float32),
        pltpu.SemaphoreType.DMA,
        pltpu.SemaphoreType.DMA,
        pltpu.SemaphoreType.DMA,
        pltpu.SemaphoreType.DMA,
    ],
)
def _sc_spmm(table, pk, zeros, out,
             pk0, pk1, rows0, rows1, acc, sg0, sg1, ss0, ss1):
    cid = lax.axis_index("c")
    sid = lax.axis_index("s")
    wid = cid * NS + sid
    rpt = N_ROWS // NS
    base = wid * CHUNKS_PER_TILE

    pltpu.sync_copy(zeros.at[pl.ds(sid * rpt, rpt)],
                    acc.at[pl.ds(sid * rpt, rpt)])
    plsc.subcore_barrier()

    pks = (pk0, pk1)
    rowss = (rows0, rows1)
    sgs = (sg0, sg1)
    sss = (ss0, ss1)

    def scale(pkb, rowsb):
        for g in range(K // 16):
            vv = plsc.bitcast(pkb[2, pl.ds(g * 16, 16)], jnp.float32)
            for rr in range(16):
                r = g * 16 + rr
                bv = lax.gather(
                    vv, jnp.full((16, 1), rr, jnp.int32),
                    lax.GatherDimensionNumbers(
                        offset_dims=(), collapsed_slice_dims=(0,),
                        start_index_map=(0,)),
                    slice_sizes=(1,),
                    mode=lax.GatherScatterMode.PROMISE_IN_BOUNDS)
                for j in range(D // 16):
                    sl = pl.ds(j * 16, 16)
                    rowsb[r, sl] = rowsb[r, sl] * bv

    pltpu.sync_copy(pk.at[base], pk0)
    pltpu.async_copy(table.at[pk0.at[0]], rows0, sg0)

    def body(jj, carry):
        for b in range(2):
            t = jj * 2 + b
            o = 1 - b
            pltpu.make_async_copy(zeros.at[pl.ds(0, K)], rowss[b], sgs[b]).wait()

            @pl.when(t > 0)
            def _():
                pltpu.make_async_copy(zeros.at[pl.ds(0, K)], rowss[o], sss[o]).wait()

            @pl.when(t + 1 < CHUNKS_PER_TILE)
            def _():
                pltpu.sync_copy(pk.at[base + t + 1], pks[o])
                pltpu.async_copy(table.at[pks[o].at[0]], rowss[o], sgs[o])

            scale(pks[b], rowss[b])
            pltpu.async_copy(rowss[b], acc.at[pks[b].at[1]], sss[b], add=True)
        return carry

    lax.fori_loop(0, CHUNKS_PER_TILE // 2, body, 0)
    pltpu.make_async_copy(zeros.at[pl.ds(0, K)], rows1, ss1).wait()
    plsc.subcore_barrier()
    pltpu.sync_copy(acc.at[pl.ds(sid * rpt, rpt)], out.at[wid])


_BLK = 1024


def _div_body(sino_ref, pp_ref, out_ref):
    out_ref[...] = sino_ref[...] / (pp_ref[0] + pp_ref[1] + 1e-8)


def _temp_proj(sinogram, pp):
    return pl.pallas_call(
        _div_body,
        grid=(N_ROWS // _BLK,),
        in_specs=[
            pl.BlockSpec((_BLK, D), lambda i: (i, 0)),
            pl.BlockSpec((2, _BLK, D), lambda i: (0, i, 0)),
        ],
        out_specs=pl.BlockSpec((_BLK, D), lambda i: (i, 0)),
        out_shape=jax.ShapeDtypeStruct((N_ROWS, D), jnp.float32),
    )(sinogram, pp)


def _final_body(img_ref, eff_ref, pp_ref, out_ref):
    out_ref[...] = img_ref[...] / eff_ref[...] * (pp_ref[0] + pp_ref[1])


def _final(image, eff, pp):
    return pl.pallas_call(
        _final_body,
        grid=(N_COLS // _BLK,),
        in_specs=[
            pl.BlockSpec((_BLK, D), lambda i: (i, 0)),
            pl.BlockSpec((_BLK, D), lambda i: (i, 0)),
            pl.BlockSpec((2, _BLK, D), lambda i: (0, i, 0)),
        ],
        out_specs=pl.BlockSpec((_BLK, D), lambda i: (i, 0)),
        out_shape=jax.ShapeDtypeStruct((N_COLS, D), jnp.float32),
    )(image, eff, pp)


def kernel(image, efficiency_map, sinogram, matrix_vals, matrix_rows, matrix_cols):
    pad = NNZ_PAD - NNZ
    cols2 = jnp.concatenate([matrix_cols, jnp.zeros((pad,), jnp.int32)]).reshape(M, K)
    rows2 = jnp.concatenate([matrix_rows, jnp.zeros((pad,), jnp.int32)]).reshape(M, K)
    valsb = jax.lax.bitcast_convert_type(
        jnp.concatenate([matrix_vals, jnp.zeros((pad,), jnp.float32)]),
        jnp.int32).reshape(M, K)
    pk_f = jnp.stack([cols2, rows2, valsb], axis=1)
    pk_b = jnp.stack([rows2, cols2, valsb], axis=1)
    zeros = jnp.zeros((N_ROWS, D), jnp.float32)

    pp = _sc_spmm(image, pk_f, zeros).reshape(NC, N_ROWS, D)
    temp = _temp_proj(sinogram, pp)
    bp = _sc_spmm(temp, pk_b, zeros).reshape(NC, N_COLS, D)
    return _final(image, efficiency_map, bp)


# speedup vs baseline: 3.7368x; 1.6908x over previous
import functools

import jax
import jax.numpy as jnp
from jax import lax
from jax.experimental import pallas as pl
from jax.experimental.pallas import tpu as pltpu
from jax.experimental.pallas import tpu_sc as plsc

N_ROWS = 16384
N_COLS = 16384
NNZ = 2684354
D = 64

NC = 2
NS = 16
NW = NC * NS
K = 128
CHUNKS_PER_TILE = 656
M = NW * CHUNKS_PER_TILE
NNZ_PAD = M * K

_mesh = plsc.VectorSubcoreMesh(core_axis_name="c", subcore_axis_name="s")


@functools.partial(
    pl.kernel,
    out_type=jax.ShapeDtypeStruct((NW, N_ROWS // NS, D), jnp.float32),
    mesh=_mesh,
    compiler_params=pltpu.CompilerParams(
        needs_layout_passes=False, use_tc_tiling_on_sc=False),
    scratch_types=[
        pltpu.VMEM((16, 3, K), jnp.int32),
        pltpu.VMEM((4, K, D), jnp.float32),
        pltpu.VMEM_SHARED((N_ROWS, D), jnp.float32),
        pltpu.SemaphoreType.DMA((4,)),
        pltpu.SemaphoreType.DMA((4,)),
        pltpu.SemaphoreType.DMA,
    ],
)
def _sc_spmm(table, pk, zeros, out, ring, rows, acc, sg, ss, si):
    cid = lax.axis_index("c")
    sid = lax.axis_index("s")
    wid = cid * NS + sid
    rpt = N_ROWS // NS
    base = wid * CHUNKS_PER_TILE
    CPT = CHUNKS_PER_TILE

    pltpu.sync_copy(zeros.at[pl.ds(sid * rpt, rpt)],
                    acc.at[pl.ds(sid * rpt, rpt)])
    plsc.subcore_barrier()

    def scale(s, b):
        for g in range(K // 16):
            vv = plsc.bitcast(ring[s, 2, pl.ds(g * 16, 16)], jnp.float32)
            for rr in range(16):
                r = g * 16 + rr
                bv = lax.gather(
                    vv, jnp.full((16, 1), rr, jnp.int32),
                    lax.GatherDimensionNumbers(
                        offset_dims=(), collapsed_slice_dims=(0,),
                        start_index_map=(0,)),
                    slice_sizes=(1,),
                    mode=lax.GatherScatterMode.PROMISE_IN_BOUNDS)
                for j in range(D // 16):
                    sl = pl.ds(j * 16, 16)
                    rows[b, r, sl] = rows[b, r, sl] * bv

    pltpu.sync_copy(pk.at[pl.ds(base, 8)], ring.at[pl.ds(0, 8)])
    pltpu.async_copy(table.at[ring.at[0, 0]], rows.at[0], sg.at[0])
    pltpu.async_copy(table.at[ring.at[1, 0]], rows.at[1], sg.at[1])

    def body(v, carry):
        b = v % 4
        s = v % 16
        pltpu.make_async_copy(zeros.at[pl.ds(0, K)], rows.at[b], sg.at[b]).wait()
        scale(s, b)
        pltpu.async_copy(rows.at[b], acc.at[ring.at[s, 1]], ss.at[b], add=True)

        @pl.when(v >= 2)
        def _():
            b2 = (v - 2) % 4
            pltpu.make_async_copy(zeros.at[pl.ds(0, K)], rows.at[b2],
                                  ss.at[b2]).wait()

        @pl.when(jnp.logical_and(v % 8 == 2, v + 6 < CPT))
        def _():
            slot = ((v + 6) // 8 % 2) * 8
            pltpu.async_copy(pk.at[pl.ds(base + v + 6, 8)],
                             ring.at[pl.ds(slot, 8)], si)

        @pl.when(jnp.logical_and(v % 8 == 6, v + 2 < CPT))
        def _():
            pltpu.make_async_copy(pk.at[pl.ds(base, 8)],
                                  ring.at[pl.ds(0, 8)], si).wait()

        @pl.when(v + 2 < CPT)
        def _():
            s2 = (v + 2) % 16
            b3 = (v + 2) % 4
            pltpu.async_copy(table.at[ring.at[s2, 0]], rows.at[b3], sg.at[b3])
        return carry

    lax.fori_loop(0, CPT, body, 0)
    pltpu.make_async_copy(zeros.at[pl.ds(0, K)], rows.at[(CPT - 2) % 4],
                          ss.at[(CPT - 2) % 4]).wait()
    pltpu.make_async_copy(zeros.at[pl.ds(0, K)], rows.at[(CPT - 1) % 4],
                          ss.at[(CPT - 1) % 4]).wait()
    plsc.subcore_barrier()
    pltpu.sync_copy(acc.at[pl.ds(sid * rpt, rpt)], out.at[wid])


_BLK = 1024


def _div_body(sino_ref, pp_ref, out_ref):
    out_ref[...] = sino_ref[...] / (pp_ref[0] + pp_ref[1] + 1e-8)


def _temp_proj(sinogram, pp):
    return pl.pallas_call(
        _div_body,
        grid=(N_ROWS // _BLK,),
        in_specs=[
            pl.BlockSpec((_BLK, D), lambda i: (i, 0)),
            pl.BlockSpec((2, _BLK, D), lambda i: (0, i, 0)),
        ],
        out_specs=pl.BlockSpec((_BLK, D), lambda i: (i, 0)),
        out_shape=jax.ShapeDtypeStruct((N_ROWS, D), jnp.float32),
    )(sinogram, pp)


def _final_body(img_ref, eff_ref, pp_ref, out_ref):
    out_ref[...] = img_ref[...] / eff_ref[...] * (pp_ref[0] + pp_ref[1])


def _final(image, eff, pp):
    return pl.pallas_call(
        _final_body,
        grid=(N_COLS // _BLK,),
        in_specs=[
            pl.BlockSpec((_BLK, D), lambda i: (i, 0)),
            pl.BlockSpec((_BLK, D), lambda i: (i, 0)),
            pl.BlockSpec((2, _BLK, D), lambda i: (0, i, 0)),
        ],
        out_specs=pl.BlockSpec((_BLK, D), lambda i: (i, 0)),
        out_shape=jax.ShapeDtypeStruct((N_COLS, D), jnp.float32),
    )(image, eff, pp)


def kernel(image, efficiency_map, sinogram, matrix_vals, matrix_rows, matrix_cols):
    pad = NNZ_PAD - NNZ
    cols2 = jnp.concatenate([matrix_cols, jnp.zeros((pad,), jnp.int32)]).reshape(M, K)
    rows2 = jnp.concatenate([matrix_rows, jnp.zeros((pad,), jnp.int32)]).reshape(M, K)
    valsb = jax.lax.bitcast_convert_type(
        jnp.concatenate([matrix_vals, jnp.zeros((pad,), jnp.float32)]),
        jnp.int32).reshape(M, K)
    pk_f = jnp.stack([cols2, rows2, valsb], axis=1)
    pk_b = jnp.stack([rows2, cols2, valsb], axis=1)
    zeros = jnp.zeros((N_ROWS, D), jnp.float32)

    pp = _sc_spmm(image, pk_f, zeros).reshape(NC, N_ROWS, D)
    temp = _temp_proj(sinogram, pp)
    bp = _sc_spmm(temp, pk_b, zeros).reshape(NC, N_COLS, D)
    return _final(image, efficiency_map, bp)


# speedup vs baseline: 4.2851x; 1.1467x over previous
import functools

import jax
import jax.numpy as jnp
from jax import lax
from jax.experimental import pallas as pl
from jax.experimental.pallas import tpu as pltpu
from jax.experimental.pallas import tpu_sc as plsc

N_ROWS = 16384
N_COLS = 16384
NNZ = 2684354
D = 64

NC = 2
NS = 16
NW = NC * NS
K = 128
CHUNKS_PER_TILE = 656
M = NW * CHUNKS_PER_TILE
NNZ_PAD = M * K

_mesh = plsc.VectorSubcoreMesh(core_axis_name="c", subcore_axis_name="s")


@functools.partial(
    pl.kernel,
    out_type=jax.ShapeDtypeStruct((NW, N_ROWS // NS, D), jnp.float32),
    mesh=_mesh,
    compiler_params=pltpu.CompilerParams(
        needs_layout_passes=False, use_tc_tiling_on_sc=False),
    scratch_types=[
        pltpu.VMEM((24, 3, K), jnp.int32),
        pltpu.VMEM((6, K, D), jnp.float32),
        pltpu.VMEM_SHARED((N_ROWS, D), jnp.float32),
        pltpu.SemaphoreType.DMA((6,)),
        pltpu.SemaphoreType.DMA((6,)),
        pltpu.SemaphoreType.DMA,
    ],
)
def _sc_spmm(table, pk, zeros, out, ring, rows, acc, sg, ss, si):
    cid = lax.axis_index("c")
    sid = lax.axis_index("s")
    wid = cid * NS + sid
    rpt = N_ROWS // NS
    base = wid * CHUNKS_PER_TILE
    CPT = CHUNKS_PER_TILE

    pltpu.sync_copy(zeros.at[pl.ds(sid * rpt, rpt)],
                    acc.at[pl.ds(sid * rpt, rpt)])
    plsc.subcore_barrier()

    def scale(s, b):
        @functools.partial(plsc.parallel_loop, 0, K // 16, unroll=K // 16)
        def _(g):
            vv = plsc.bitcast(ring[s, 2, pl.ds(g * 16, 16)], jnp.float32)
            for rr in range(16):
                bv = lax.gather(
                    vv, jnp.full((16, 1), rr, jnp.int32),
                    lax.GatherDimensionNumbers(
                        offset_dims=(), collapsed_slice_dims=(0,),
                        start_index_map=(0,)),
                    slice_sizes=(1,),
                    mode=lax.GatherScatterMode.PROMISE_IN_BOUNDS)
                r = g * 16 + rr
                for j in range(D // 16):
                    sl = pl.ds(j * 16, 16)
                    rows[b, r, sl] = rows[b, r, sl] * bv

    pltpu.sync_copy(pk.at[pl.ds(base, 16)], ring.at[pl.ds(0, 16)])
    for c0 in range(3):
        pltpu.async_copy(table.at[ring.at[c0, 0]], rows.at[c0], sg.at[c0])

    def body(v, carry):
        b = v % 6
        s = v % 24
        pltpu.make_async_copy(zeros.at[pl.ds(0, K)], rows.at[b], sg.at[b]).wait()
        scale(s, b)
        pltpu.async_copy(rows.at[b], acc.at[ring.at[s, 1]], ss.at[b], add=True)

        @pl.when(v >= 3)
        def _():
            b2 = (v - 3) % 6
            pltpu.make_async_copy(zeros.at[pl.ds(0, K)], rows.at[b2],
                                  ss.at[b2]).wait()

        @pl.when(jnp.logical_and(v % 8 == 5, jnp.logical_and(v + 3 < CPT, v >= 13)))
        def _():
            pltpu.make_async_copy(pk.at[pl.ds(base, 8)],
                                  ring.at[pl.ds(0, 8)], si).wait()

        @pl.when(jnp.logical_and(v % 8 == 5, v + 11 < CPT))
        def _():
            slot = ((v + 11) // 8 % 3) * 8
            pltpu.async_copy(pk.at[pl.ds(base + v + 11, 8)],
                             ring.at[pl.ds(slot, 8)], si)

        @pl.when(v + 3 < CPT)
        def _():
            s2 = (v + 3) % 24
            b3 = (v + 3) % 6
            pltpu.async_copy(table.at[ring.at[s2, 0]], rows.at[b3], sg.at[b3])
        return carry

    lax.fori_loop(0, CPT, body, 0)
    for c0 in range(3):
        b4 = (CPT - 3 + c0) % 6
        pltpu.make_async_copy(zeros.at[pl.ds(0, K)], rows.at[b4],
                              ss.at[b4]).wait()
    plsc.subcore_barrier()
    pltpu.sync_copy(acc.at[pl.ds(sid * rpt, rpt)], out.at[wid])


_BLK = 1024


def _div_body(sino_ref, pp_ref, out_ref):
    out_ref[...] = sino_ref[...] / (pp_ref[0] + pp_ref[1] + 1e-8)


def _temp_proj(sinogram, pp):
    return pl.pallas_call(
        _div_body,
        grid=(N_ROWS // _BLK,),
        in_specs=[
            pl.BlockSpec((_BLK, D), lambda i: (i, 0)),
            pl.BlockSpec((2, _BLK, D), lambda i: (0, i, 0)),
        ],
        out_specs=pl.BlockSpec((_BLK, D), lambda i: (i, 0)),
        out_shape=jax.ShapeDtypeStruct((N_ROWS, D), jnp.float32),
    )(sinogram, pp)


def _final_body(img_ref, eff_ref, pp_ref, out_ref):
    out_ref[...] = img_ref[...] / eff_ref[...] * (pp_ref[0] + pp_ref[1])


def _final(image, eff, pp):
    return pl.pallas_call(
        _final_body,
        grid=(N_COLS // _BLK,),
        in_specs=[
            pl.BlockSpec((_BLK, D), lambda i: (i, 0)),
            pl.BlockSpec((_BLK, D), lambda i: (i, 0)),
            pl.BlockSpec((2, _BLK, D), lambda i: (0, i, 0)),
        ],
        out_specs=pl.BlockSpec((_BLK, D), lambda i: (i, 0)),
        out_shape=jax.ShapeDtypeStruct((N_COLS, D), jnp.float32),
    )(image, eff, pp)


def kernel(image, efficiency_map, sinogram, matrix_vals, matrix_rows, matrix_cols):
    pad = NNZ_PAD - NNZ
    cols2 = jnp.concatenate([matrix_cols, jnp.zeros((pad,), jnp.int32)]).reshape(M, K)
    rows2 = jnp.concatenate([matrix_rows, jnp.zeros((pad,), jnp.int32)]).reshape(M, K)
    valsb = jax.lax.bitcast_convert_type(
        jnp.concatenate([matrix_vals, jnp.zeros((pad,), jnp.float32)]),
        jnp.int32).reshape(M, K)
    pk_f = jnp.stack([cols2, rows2, valsb], axis=1)
    pk_b = jnp.stack([rows2, cols2, valsb], axis=1)
    zeros = jnp.zeros((N_ROWS, D), jnp.float32)

    pp = _sc_spmm(image, pk_f, zeros).reshape(NC, N_ROWS, D)
    temp = _temp_proj(sinogram, pp)
    bp = _sc_spmm(temp, pk_b, zeros).reshape(NC, N_COLS, D)
    return _final(image, efficiency_map, bp)
